# baseline (speedup 1.0000x reference)
import numpy as np
import jax
import jax.numpy as jnp
from jax import lax
from jax.experimental import pallas as pl
from jax.experimental.pallas import tpu as pltpu
from jax.experimental.pallas import tpu_sc as plsc

_N = 30
_NC2 = _N * (_N - 1) // 2
_PLANE = _N * _NC2
_PAD = 448
_NCHUNK = _PAD // 16
_TAIL = _NC2 - (_NCHUNK - 1) * 16
_K = 4
_NSLOT = 2


def _build_flat_idx():
    ia = np.zeros((_PAD,), dtype=np.int32)
    ib = np.zeros((_PAD,), dtype=np.int32)
    p = 0
    for i2 in range(_N):
        for j2 in range(i2):
            ia[p] = i2 * _NC2 + p
            ib[p] = j2 * _NC2 + p
            p += 1
    return ia, ib


_IDX_A, _IDX_B = _build_flat_idx()


def _scatter_row(nrf_buf, k, ia_v, ib_v, plane, off):
    lane = lax.iota(jnp.int32, 16)
    for j in range(_NCHUNK):
        sl = pl.ds(k * _PAD + j * 16, 16)
        isl = pl.ds(j * 16, 16)
        v = nrf_buf[sl] * 0.5
        ia = ia_v[isl] + off
        ib = ib_v[isl] + off
        if j == _NCHUNK - 1:
            mask = lane < _TAIL
            plsc.store_scatter(plane, [ia], v, mask=mask)
            plsc.store_scatter(plane, [ib], v, mask=mask)
        else:
            plsc.store_scatter(plane, [ia], v)
            plsc.store_scatter(plane, [ib], v)


def _sc_body(nrf_hbm, ia_hbm, ib_hbm, zeros_hbm, out_hbm,
             nrf0, nrf1, ia_v, ib_v, pg0, pg1,
             sem_in0, sem_in1, sem_out0, sem_out1):
    nrfs = (nrf0, nrf1)
    groups_bufs = (pg0, pg1)
    sems_in = (sem_in0, sem_in1)
    sems_out = (sem_out0, sem_out1)

    num_cores = lax.axis_size("c")
    num_sub = lax.axis_size("s")
    wid = lax.axis_index("s") * num_cores + lax.axis_index("c")
    nw = num_cores * num_sub
    batch = nrf_hbm.shape[0] // _PAD
    rows = batch // nw
    base = wid * rows
    groups = rows // _K

    pltpu.sync_copy(ia_hbm, ia_v)
    pltpu.sync_copy(ib_hbm, ib_v)
    for q in range(_NSLOT):
        pltpu.sync_copy(zeros_hbm, groups_bufs[q])
        pltpu.async_copy(nrf_hbm.at[pl.ds((base + q * _K) * _PAD, _K * _PAD)],
                         nrfs[q], sems_in[q])

    def step(gg, carry):
        for q in range(_NSLOT):
            g = gg * _NSLOT + q
            b = base + g * _K
            @pl.when(gg >= 1)
            def _():
                pltpu.make_async_copy(
                    groups_bufs[q],
                    out_hbm.at[pl.ds(b * _PLANE, _K * _PLANE)],
                    sems_out[q]).wait()
            pltpu.make_async_copy(nrf_hbm.at[pl.ds(b * _PAD, _K * _PAD)],
                                  nrfs[q], sems_in[q]).wait()
            for k in range(_K):
                _scatter_row(nrfs[q], k, ia_v, ib_v, groups_bufs[q],
                             k * _PLANE)
            @pl.when(gg < (groups // _NSLOT) - 1)
            def _():
                pltpu.async_copy(
                    nrf_hbm.at[pl.ds((b + _NSLOT * _K) * _PAD, _K * _PAD)],
                    nrfs[q], sems_in[q])
            pltpu.async_copy(groups_bufs[q],
                             out_hbm.at[pl.ds(b * _PLANE, _K * _PLANE)],
                             sems_out[q])
        return carry

    lax.fori_loop(0, groups // _NSLOT, step, 0)

    for q in range(_NSLOT):
        b = base + (groups - _NSLOT + q) * _K
        pltpu.make_async_copy(groups_bufs[q],
                              out_hbm.at[pl.ds(b * _PLANE, _K * _PLANE)],
                              sems_out[q]).wait()


def kernel(_NRF):
    b = _NRF.shape[0]
    mesh = plsc.VectorSubcoreMesh(core_axis_name="c", subcore_axis_name="s")
    sc_call = pl.kernel(
        _sc_body,
        out_type=jax.ShapeDtypeStruct((b * _PLANE,), _NRF.dtype),
        mesh=mesh,
        scratch_types=[
            pltpu.VMEM((_K * _PAD,), jnp.float32),
            pltpu.VMEM((_K * _PAD,), jnp.float32),
            pltpu.VMEM((_PAD,), jnp.int32),
            pltpu.VMEM((_PAD,), jnp.int32),
            pltpu.VMEM((_K * _PLANE,), jnp.float32),
            pltpu.VMEM((_K * _PLANE,), jnp.float32),
            pltpu.SemaphoreType.DMA,
            pltpu.SemaphoreType.DMA,
            pltpu.SemaphoreType.DMA,
            pltpu.SemaphoreType.DMA,
        ],
        compiler_params=pltpu.CompilerParams(
            use_tc_tiling_on_sc=False, needs_layout_passes=False),
    )
    nrf_pad = jnp.pad(_NRF, ((0, 0), (0, _PAD - _NC2))).reshape(-1)
    flat = sc_call(
        nrf_pad,
        jnp.asarray(_IDX_A),
        jnp.asarray(_IDX_B),
        jnp.zeros((_K * _PLANE,), jnp.float32),
    )
    return flat.reshape(b, _N, _NC2)


# speedup vs baseline: 2.1576x; 2.1576x over previous
import numpy as np
import jax
import jax.numpy as jnp
from jax import lax
from jax.experimental import pallas as pl
from jax.experimental.pallas import tpu as pltpu
from jax.experimental.pallas import tpu_sc as plsc

_N = 30
_NC2 = _N * (_N - 1) // 2
_PAD = 448
_NCHUNK = _PAD // 16
_TAIL = _NC2 - (_NCHUNK - 1) * 16
_NBUF = 2


def _build_pair_rows():
    ra = np.zeros((_PAD,), dtype=np.int32)
    rb = np.zeros((_PAD,), dtype=np.int32)
    p = 0
    for i2 in range(_N):
        for j2 in range(i2):
            ra[p] = i2
            rb[p] = j2
            p += 1
    return ra, rb


_ROW_A, _ROW_B = _build_pair_rows()


def _scatter_row(nrf_v, ra_v, rb_v, plane):
    lane = lax.iota(jnp.int32, 16)
    for j in range(_NCHUNK):
        sl = pl.ds(j * 16, 16)
        v = nrf_v[sl] * 0.5
        col = lane + (j * 16)
        ra = ra_v[sl]
        rb = rb_v[sl]
        if j == _NCHUNK - 1:
            mask = lane < _TAIL
            plsc.store_scatter(plane, [ra, col], v, mask=mask)
            plsc.store_scatter(plane, [rb, col], v, mask=mask)
        else:
            plsc.store_scatter(plane, [ra, col], v)
            plsc.store_scatter(plane, [rb, col], v)


def _sc_body(nrf_hbm, ra_hbm, rb_hbm, zeros_hbm, out_hbm, *scratch):
    nrfs = scratch[0:_NBUF]
    ra_v = scratch[_NBUF]
    rb_v = scratch[_NBUF + 1]
    planes = scratch[_NBUF + 2:2 * _NBUF + 2]
    sems_in = scratch[2 * _NBUF + 2:3 * _NBUF + 2]
    sems_out = scratch[3 * _NBUF + 2:4 * _NBUF + 2]

    num_cores = lax.axis_size("c")
    num_sub = lax.axis_size("s")
    wid = lax.axis_index("s") * num_cores + lax.axis_index("c")
    nw = num_cores * num_sub
    batch = nrf_hbm.shape[0]
    rows = batch // nw
    base = wid * rows

    pltpu.sync_copy(ra_hbm, ra_v)
    pltpu.sync_copy(rb_hbm, rb_v)
    for q in range(_NBUF):
        pltpu.sync_copy(zeros_hbm, planes[q])
        pltpu.async_copy(nrf_hbm.at[base + q],
                         nrfs[q].at[pl.ds(0, _NC2)], sems_in[q])

    def step(rr, carry):
        for q in range(_NBUF):
            r = rr * _NBUF + q
            b = base + r
            @pl.when(rr >= 1)
            def _():
                pltpu.make_async_copy(planes[q], out_hbm.at[b],
                                      sems_out[q]).wait()
            pltpu.make_async_copy(nrf_hbm.at[b],
                                  nrfs[q].at[pl.ds(0, _NC2)],
                                  sems_in[q]).wait()
            _scatter_row(nrfs[q], ra_v, rb_v, planes[q])
            @pl.when(rr < (rows // _NBUF) - 1)
            def _():
                pltpu.async_copy(nrf_hbm.at[b + _NBUF],
                                 nrfs[q].at[pl.ds(0, _NC2)], sems_in[q])
            pltpu.async_copy(planes[q], out_hbm.at[b], sems_out[q])
        return carry

    lax.fori_loop(0, rows // _NBUF, step, 0)

    for q in range(_NBUF):
        pltpu.make_async_copy(planes[q],
                              out_hbm.at[base + rows - _NBUF + q],
                              sems_out[q]).wait()


def kernel(_NRF):
    b = _NRF.shape[0]
    mesh = plsc.VectorSubcoreMesh(core_axis_name="c", subcore_axis_name="s")
    scratch = (
        [pltpu.VMEM((_PAD,), jnp.float32) for _ in range(_NBUF)]
        + [pltpu.VMEM((_PAD,), jnp.int32) for _ in range(2)]
        + [pltpu.VMEM((_N, _NC2), jnp.float32) for _ in range(_NBUF)]
        + [pltpu.SemaphoreType.DMA for _ in range(2 * _NBUF)]
    )
    sc_call = pl.kernel(
        _sc_body,
        out_type=jax.ShapeDtypeStruct((b, _N, _NC2), _NRF.dtype),
        mesh=mesh,
        scratch_types=scratch,
        compiler_params=pltpu.CompilerParams(
            use_tc_tiling_on_sc=False, needs_layout_passes=False),
    )
    return sc_call(
        _NRF,
        jnp.asarray(_ROW_A),
        jnp.asarray(_ROW_B),
        jnp.zeros((_N, _NC2), jnp.float32),
    )
